# baseline (device time: 21683 ns/iter reference)
import os

import jax
import jax.numpy as jnp
from jax import lax
from jax.experimental import pallas as pl
from jax.experimental.pallas import tpu as pltpu

_VARIANT = os.environ.get("SCB_VARIANT", "full")

T = 512
D = 512
V_PER = 4096
N_X = 2
N_Y = 4
N_Z = 4
N_S = N_X * N_Z
R = T // N_S


def kernel(ids, E):
    ids2 = ids.reshape(T, 1)

    def body(idv_ref, e_ref, out_ref, yb, fb, ys, yr, fs, fr):
        my_x = lax.axis_index("x")
        my_y = lax.axis_index("y")
        my_z = lax.axis_index("z")
        my_s = my_x * N_Z + my_z
        row0 = my_s * R

        def xz_peers():
            peers = [(1 - my_x, my_y, my_z)]
            for k in range(1, N_Z):
                peers.append((my_x, my_y, (my_z + k) % N_Z))
                peers.append((1 - my_x, my_y, (my_z + k) % N_Z))
            return peers

        if _VARIANT != "compute":
            barrier_sem = pltpu.get_barrier_semaphore()
            for k in range(1, N_Y):
                pl.semaphore_signal(
                    barrier_sem, inc=1,
                    device_id=(my_x, (my_y + k) % N_Y, my_z),
                    device_id_type=pl.DeviceIdType.MESH,
                )
            for p in xz_peers():
                pl.semaphore_signal(
                    barrier_sem, inc=1,
                    device_id=p, device_id_type=pl.DeviceIdType.MESH,
                )

        if _VARIANT != "comm":
            lid = idv_ref[pl.ds(row0, R), :] - my_y * V_PER
            iota = lax.broadcasted_iota(jnp.int32, (R, V_PER), 1)
            onehot = (iota == lid).astype(jnp.float32)
            p = jnp.dot(
                onehot, e_ref[:, :], preferred_element_type=jnp.float32
            )
            yb[my_y] = p.astype(jnp.bfloat16)
        else:
            yb[my_y] = jnp.zeros((R, D), jnp.bfloat16)
        if _VARIANT == "compute":
            out_ref[:, :] = jnp.zeros((T, D), jnp.bfloat16)
            out_ref[pl.ds(row0, R), :] = yb[my_y]
            return

        pl.semaphore_wait(barrier_sem, 10)

        y_sends = []
        for k in range(1, N_Y):
            r = pltpu.make_async_remote_copy(
                src_ref=yb.at[my_y], dst_ref=yb.at[my_y],
                send_sem=ys.at[k - 1], recv_sem=yr.at[my_y],
                device_id=(my_x, (my_y + k) % N_Y, my_z),
                device_id_type=pl.DeviceIdType.MESH,
            )
            r.start()
            y_sends.append(r)
        for k in range(1, N_Y):
            src_y = (my_y + k) % N_Y
            pltpu.make_async_remote_copy(
                src_ref=yb.at[my_y], dst_ref=yb.at[src_y],
                send_sem=ys.at[k - 1], recv_sem=yr.at[src_y],
                device_id=(my_x, src_y, my_z),
                device_id_type=pl.DeviceIdType.MESH,
            ).wait_recv()
        fin = yb[0] + yb[1] + yb[2] + yb[3]
        fb[my_s] = fin

        f_sends = []
        for i, p in enumerate(xz_peers()):
            r = pltpu.make_async_remote_copy(
                src_ref=fb.at[my_s], dst_ref=fb.at[my_s],
                send_sem=fs.at[i], recv_sem=fr.at[my_s],
                device_id=p, device_id_type=pl.DeviceIdType.MESH,
            )
            r.start()
            f_sends.append(r)
        out_ref[pl.ds(row0, R), :] = fin
        for i, p in enumerate(xz_peers()):
            src_s = p[0] * N_Z + p[2]
            pltpu.make_async_remote_copy(
                src_ref=fb.at[my_s], dst_ref=fb.at[src_s],
                send_sem=fs.at[i], recv_sem=fr.at[src_s],
                device_id=p, device_id_type=pl.DeviceIdType.MESH,
            ).wait_recv()
            out_ref[pl.ds(src_s * R, R), :] = fb[src_s]

        for r in y_sends + f_sends:
            r.wait_send()

    return pl.pallas_call(
        body,
        out_shape=jax.ShapeDtypeStruct((T, D), jnp.bfloat16),
        in_specs=[
            pl.BlockSpec(memory_space=pltpu.VMEM),
            pl.BlockSpec(memory_space=pltpu.VMEM),
        ],
        out_specs=pl.BlockSpec(memory_space=pltpu.VMEM),
        scratch_shapes=[
            pltpu.VMEM((N_Y, R, D), jnp.bfloat16),
            pltpu.VMEM((N_S, R, D), jnp.bfloat16),
            pltpu.SemaphoreType.DMA((N_Y - 1,)),
            pltpu.SemaphoreType.DMA((N_Y,)),
            pltpu.SemaphoreType.DMA((N_S - 1,)),
            pltpu.SemaphoreType.DMA((N_S,)),
        ],
        compiler_params=(
            pltpu.CompilerParams(collective_id=0)
            if _VARIANT != "compute"
            else pltpu.CompilerParams()
        ),
    )(ids2, E)
